# baseline (speedup 1.0000x reference)
import jax
import jax.numpy as jnp
from jax import lax
from jax.experimental import pallas as pl
from jax.experimental.pallas import tpu as pltpu
from jax.experimental.pallas import tpu_sc as plsc

B, C, H, W = 8, 192, 224, 224
HW = H * W
NS = B * C
NW = 32
CPW = NS // NW
WPB = C // CPW
DEPTH = 8


def _shuffle_body(x_hbm, perm_hbm, out_hbm, pbuf, sem):
    cid = lax.axis_index("c")
    sid = lax.axis_index("s")
    wid = sid * 2 + cid
    b = wid // WPB
    cbase = (wid % WPB) * CPW

    pltpu.sync_copy(perm_hbm.at[pl.ds(pl.multiple_of(cbase, 8), CPW)], pbuf)

    def wait_one():
        pltpu.make_async_copy(x_hbm.at[0], out_hbm.at[0], sem).wait()

    inflight = 0
    for g in range(CPW // 16):
        pv = pbuf[pl.ds(16 * g, 16)]
        for l in range(16):
            j = 16 * g + l
            src = b * C + pv[l]
            dst = b * C + cbase + j
            pltpu.async_copy(x_hbm.at[src], out_hbm.at[dst], sem)
            inflight += 1
            if inflight > DEPTH:
                wait_one()
                inflight -= 1
    for _ in range(inflight):
        wait_one()


@jax.jit
def _shuffle(x2, perm):
    mesh = plsc.VectorSubcoreMesh(core_axis_name="c", subcore_axis_name="s")
    return pl.kernel(
        _shuffle_body,
        out_type=jax.ShapeDtypeStruct((NS, HW), jnp.float32),
        mesh=mesh,
        scratch_types=[
            pltpu.VMEM((CPW,), jnp.int32),
            pltpu.SemaphoreType.DMA,
        ],
    )(x2, perm)


def kernel(x, perm):
    x2 = x.reshape(NS, HW)
    y2 = _shuffle(x2, perm.astype(jnp.int32))
    return (y2.reshape(B, C, H, W), jnp.zeros((), dtype=jnp.float32))


# speedup vs baseline: 10.1451x; 10.1451x over previous
import jax
import jax.numpy as jnp
from jax import lax
from jax.experimental import pallas as pl
from jax.experimental.pallas import tpu as pltpu
from jax.experimental.pallas import tpu_sc as plsc

B, C, H, W = 8, 192, 224, 224
HW = H * W
G = 8
RL = HW // G
NROWS = B * C * G
NW = 32
CPW = (B * C) // NW
WPB = C // CPW


def _shuffle_body(x_hbm, perm_hbm, out_hbm, pbuf, idx_v, buf0, buf1,
                  g0, g1, s0, s1):
    cid = lax.axis_index("c")
    sid = lax.axis_index("s")
    wid = sid * 2 + cid
    b = wid // WPB
    cbase = (wid % WPB) * CPW

    pltpu.sync_copy(perm_hbm.at[pl.ds(pl.multiple_of(cbase, 8), CPW)], pbuf)

    lanes = lax.iota(jnp.int32, 16)
    sub = lanes & (G - 1)
    for gblk in range(CPW // 16):
        pv = pbuf[pl.ds(16 * gblk, 16)]
        for half in range(8):
            v = 8 * gblk + half
            sa = (b * C + pv[2 * half]) * G
            sb = (b * C + pv[2 * half + 1]) * G
            idx_v[pl.ds(16 * v, 16)] = jnp.where(lanes < 8, sa, sb) + sub

    base = wid * CPW

    def fire_gather(i, buf, sem):
        src = x_hbm.at[idx_v.at[pl.ds(pl.multiple_of(8 * i, 8), 8)]]
        return pltpu.async_copy(src, buf, sem)

    def fire_scatter(i, buf, sem):
        dst = out_hbm.at[pl.ds(pl.multiple_of((base + i) * G, 8), G)]
        pltpu.async_copy(buf, dst, sem)

    def wait_scatter(sem):
        pltpu.make_async_copy(buf0, out_hbm.at[pl.ds(0, G)], sem).wait()

    fire_gather(0, buf0, g0).wait()
    fire_scatter(0, buf0, s0)
    fire_gather(1, buf1, g1).wait()
    fire_scatter(1, buf1, s1)

    def body(k, carry):
        i0 = 2 * k
        i1 = 2 * k + 1
        wait_scatter(s0)
        gd0 = fire_gather(i0, buf0, g0)
        wait_scatter(s1)
        gd1 = fire_gather(i1, buf1, g1)
        gd0.wait()
        fire_scatter(i0, buf0, s0)
        gd1.wait()
        fire_scatter(i1, buf1, s1)
        return carry

    lax.fori_loop(1, CPW // 2, body, 0)
    wait_scatter(s0)
    wait_scatter(s1)


@jax.jit
def _shuffle(x2, perm):
    mesh = plsc.VectorSubcoreMesh(core_axis_name="c", subcore_axis_name="s")
    return pl.kernel(
        _shuffle_body,
        out_type=jax.ShapeDtypeStruct((NROWS, RL), jnp.float32),
        mesh=mesh,
        scratch_types=[
            pltpu.VMEM((CPW,), jnp.int32),
            pltpu.VMEM((CPW * G,), jnp.int32),
            pltpu.VMEM((G, RL), jnp.float32),
            pltpu.VMEM((G, RL), jnp.float32),
            pltpu.SemaphoreType.DMA,
            pltpu.SemaphoreType.DMA,
            pltpu.SemaphoreType.DMA,
            pltpu.SemaphoreType.DMA,
        ],
    )(x2, perm)


def kernel(x, perm):
    x2 = x.reshape(NROWS, RL)
    y2 = _shuffle(x2, perm.astype(jnp.int32))
    return (y2.reshape(B, C, H, W), jnp.zeros((), dtype=jnp.float32))
